# baseline (speedup 1.0000x reference)
import functools

import jax
import jax.numpy as jnp
from jax import lax
from jax.experimental import pallas as pl
from jax.experimental.pallas import tpu as pltpu
from jax.experimental.pallas import tpu_sc as plsc

_HIDDEN = 1024
_NUM_CORES = 2
_NUM_SUBCORES = 16
_NW = _NUM_CORES * _NUM_SUBCORES
_NBUF = 6
_CHUNK = 16


def _embed_body(b_per_w, tokens_hbm, table_hbm, out_hbm,
                idx_v, bufs, gsems, ssems):
    wid = lax.axis_index("s") * _NUM_CORES + lax.axis_index("c")
    base = wid * b_per_w
    nchunk = b_per_w // _CHUNK
    d = _NBUF
    pltpu.sync_copy(tokens_hbm.at[wid], idx_v)

    def start_gather(c, j):
        pltpu.async_copy(table_hbm.at[idx_v.at[c]], bufs[j], gsems[j])

    def wait_gather(c, j):
        pltpu.make_async_copy(
            table_hbm.at[idx_v.at[c]], bufs[j], gsems[j]).wait()

    def start_store(c, j):
        pltpu.async_copy(
            bufs[j], out_hbm.at[pl.ds(base + c * _CHUNK, _CHUNK)], ssems[j])

    def wait_store(c, j):
        pltpu.make_async_copy(
            bufs[j], out_hbm.at[pl.ds(base + c * _CHUNK, _CHUNK)],
            ssems[j]).wait()

    def process(c, j, reissue):
        wait_gather(c, j)
        start_store(c, j)
        if reissue:
            jp = (j - 1) % d
            wait_store(c - 1, jp)
            start_gather(c - 1 + d, jp)

    for j in range(d):
        start_gather(j, j)
    for c in range(d):
        process(c, c, 0 < c)

    p_hi = (nchunk - 2 * d + 1) // d

    def grp_step(p, carry):
        c0 = p * d
        for j in range(d):
            process(c0 + j, j, True)
        return carry

    lax.fori_loop(1, p_hi + 1, grp_step, 0, unroll=False)

    for c in range((p_hi + 1) * d, nchunk):
        process(c, c % d, c - 1 + d < nchunk)
    for c in range(nchunk - d, nchunk):
        wait_store(c, c % d)


def kernel(tokens, embedding):
    b = tokens.size
    b_per_w = b // _NW
    nchunk = b_per_w // _CHUNK
    flat = tokens.reshape(_NW, nchunk, _CHUNK)
    mesh = plsc.VectorSubcoreMesh(core_axis_name="c", subcore_axis_name="s")
    out = pl.kernel(
        functools.partial(_embed_body, b_per_w),
        out_type=jax.ShapeDtypeStruct((b, _HIDDEN), jnp.float32),
        mesh=mesh,
        scratch_types=[
            pltpu.VMEM((nchunk, _CHUNK), jnp.int32),
            [pltpu.VMEM((_CHUNK, _HIDDEN), jnp.float32)
             for _ in range(_NBUF)],
            [pltpu.SemaphoreType.DMA for _ in range(_NBUF)],
            [pltpu.SemaphoreType.DMA for _ in range(_NBUF)],
        ],
    )(flat, embedding)
    return out.reshape(tokens.shape + (_HIDDEN,))
